# baseline (speedup 1.0000x reference)
import functools

import jax
import jax.numpy as jnp
from jax.experimental import pallas as pl
from jax.experimental.pallas import tpu as pltpu

D_MODEL = 2048
NUM_EXPERTS = 16
NUM_SELECTED = 2
CAPACITY_FACTOR = 1.25
Z_LOSS_COEF = 0.01

TOKEN_BLOCK = 2048


def _router_block(x_ref, w_ref, idx_ref, wgt_ref, conf_ref, counts_ref,
                  lse_ref, ent_ref):
    step = pl.program_id(0)

    x = x_ref[...]
    w = w_ref[...]
    logits = jnp.dot(x, w, preferred_element_type=jnp.float32)

    m = jnp.max(logits, axis=-1, keepdims=True)
    e = jnp.exp(logits - m)
    s = jnp.sum(e, axis=-1, keepdims=True)
    probs = e / s
    lse = m[:, 0] + jnp.log(s[:, 0])

    iota = jax.lax.broadcasted_iota(jnp.int32, probs.shape, 1)

    p1 = jnp.max(probs, axis=-1, keepdims=True)
    i1 = jnp.min(jnp.where(probs == p1, iota, NUM_EXPERTS), axis=-1,
                 keepdims=True)
    probs_m = jnp.where(iota == i1, -1.0, probs)
    p2 = jnp.max(probs_m, axis=-1, keepdims=True)
    i2 = jnp.min(jnp.where(probs_m == p2, iota, NUM_EXPERTS), axis=-1,
                 keepdims=True)

    denom = p1 + p2 + 1e-8
    w1 = p1 / denom
    w2 = p2 / denom

    idx_ref[:, 0] = i1[:, 0]
    idx_ref[:, 1] = i2[:, 0]
    wgt_ref[:, 0] = w1[:, 0]
    wgt_ref[:, 1] = w2[:, 0]
    conf_ref[...] = w1[:, 0]

    hits = ((iota == i1) | (iota == i2)).astype(jnp.float32)
    block_counts = jnp.sum(hits, axis=0)[None, :]

    ent = -jnp.sum(probs * jnp.log(probs + 1e-10), axis=-1)
    block_lse = jnp.sum(lse)[None, None]
    block_ent = jnp.sum(ent)[None, None]

    @pl.when(step == 0)
    def _init():
        counts_ref[...] = block_counts
        lse_ref[...] = block_lse
        ent_ref[...] = block_ent

    @pl.when(step != 0)
    def _acc():
        counts_ref[...] += block_counts
        lse_ref[...] += block_lse
        ent_ref[...] += block_ent


@jax.jit
def _router(hidden_flat, w_t):
    n_tokens = hidden_flat.shape[0]
    grid = (n_tokens // TOKEN_BLOCK,)
    out_shapes = (
        jax.ShapeDtypeStruct((n_tokens, NUM_SELECTED), jnp.int32),
        jax.ShapeDtypeStruct((n_tokens, NUM_SELECTED), jnp.float32),
        jax.ShapeDtypeStruct((n_tokens,), jnp.float32),
        jax.ShapeDtypeStruct((1, NUM_EXPERTS), jnp.float32),
        jax.ShapeDtypeStruct((1, 1), jnp.float32),
        jax.ShapeDtypeStruct((1, 1), jnp.float32),
    )
    return pl.pallas_call(
        _router_block,
        grid=grid,
        in_specs=[
            pl.BlockSpec((TOKEN_BLOCK, D_MODEL), lambda i: (i, 0)),
            pl.BlockSpec((D_MODEL, NUM_EXPERTS), lambda i: (0, 0)),
        ],
        out_specs=(
            pl.BlockSpec((TOKEN_BLOCK, NUM_SELECTED), lambda i: (i, 0)),
            pl.BlockSpec((TOKEN_BLOCK, NUM_SELECTED), lambda i: (i, 0)),
            pl.BlockSpec((TOKEN_BLOCK,), lambda i: (i,)),
            pl.BlockSpec((1, NUM_EXPERTS), lambda i: (0, 0)),
            pl.BlockSpec((1, 1), lambda i: (0, 0)),
            pl.BlockSpec((1, 1), lambda i: (0, 0)),
        ),
        out_shape=out_shapes,
        compiler_params=pltpu.CompilerParams(
            dimension_semantics=("arbitrary",),
        ),
    )(hidden_flat, w_t)


def kernel(hidden_states, gate_weight):
    batch_size, seq_len, d_model = hidden_states.shape
    num_tokens = batch_size * seq_len
    hidden_flat = hidden_states.reshape(num_tokens, d_model)

    idx, wgt, conf, counts2d, lse_sum, ent_sum = _router(
        hidden_flat, gate_weight.T)

    expert_counts = counts2d[0]
    capacity = int(CAPACITY_FACTOR * num_tokens / NUM_EXPERTS * NUM_SELECTED)
    expert_overflow = jnp.sum(jnp.maximum(expert_counts - capacity, 0.0))
    capacity_overflow_pct = expert_overflow / num_tokens * 100.0
    z_loss = lse_sum[0, 0] / num_tokens * Z_LOSS_COEF
    gate_entropy = ent_sum[0, 0] / num_tokens
    expert_load_normalized = expert_counts / jnp.sum(expert_counts)
    ideal_load = 1.0 / NUM_EXPERTS
    expert_load_variance = jnp.mean((expert_load_normalized - ideal_load) ** 2)

    expert_indices = idx.reshape(batch_size, seq_len, NUM_SELECTED)
    expert_weights = wgt.reshape(batch_size, seq_len, NUM_SELECTED)
    routing_confidence = conf
    return (expert_indices, expert_weights, expert_counts,
            capacity_overflow_pct, z_loss, gate_entropy,
            expert_load_variance, routing_confidence)


# speedup vs baseline: 1.7468x; 1.7468x over previous
import jax
import jax.numpy as jnp
from jax import lax
from jax.experimental import pallas as pl
from jax.experimental.pallas import tpu as pltpu

D_MODEL = 2048
NUM_EXPERTS = 16
NUM_SELECTED = 2
CAPACITY_FACTOR = 1.25
Z_LOSS_COEF = 0.01

TOKEN_BLOCK = 1024
NEG_HUGE = -3.0e38


def _router_block(w_ref, x_ref, it_ref, wt_ref, counts_ref, lse_ref, ent_ref):
    step = pl.program_id(0)

    w = w_ref[...]
    x = x_ref[...]
    logits = lax.dot_general(
        w, x, dimension_numbers=(((1,), (1,)), ((), ())),
        preferred_element_type=jnp.float32)

    m = jnp.max(logits, axis=0, keepdims=True)
    e = jnp.exp(logits - m)
    s = jnp.sum(e, axis=0, keepdims=True)
    rs = 1.0 / s
    lse = m + jnp.log(s)
    sel = jnp.sum(e * logits, axis=0, keepdims=True)
    ent = lse - sel * rs

    iota = lax.broadcasted_iota(jnp.int32, logits.shape, 0)

    l1 = jnp.max(logits, axis=0, keepdims=True)
    i1 = jnp.min(jnp.where(logits == l1, iota, NUM_EXPERTS), axis=0,
                 keepdims=True)
    hit1 = iota == i1
    lm = jnp.where(hit1, NEG_HUGE, logits)
    l2 = jnp.max(lm, axis=0, keepdims=True)
    i2 = jnp.min(jnp.where(lm == l2, iota, NUM_EXPERTS), axis=0,
                 keepdims=True)
    hit2 = iota == i2

    p1 = jnp.exp(l1 - m) * rs
    p2 = jnp.exp(l2 - m) * rs
    rden = 1.0 / (p1 + p2 + 1e-8)
    w1 = p1 * rden
    w2 = p2 * rden

    it_ref[0:1, :] = i1
    it_ref[1:2, :] = i2
    wt_ref[0:1, :] = w1
    wt_ref[1:2, :] = w2

    hits = (hit1 | hit2).astype(jnp.float32)
    block_counts = jnp.sum(hits, axis=1, keepdims=True)
    block_lse = jnp.sum(lse)[None, None]
    block_ent = jnp.sum(ent)[None, None]

    @pl.when(step == 0)
    def _init():
        counts_ref[...] = block_counts
        lse_ref[...] = block_lse
        ent_ref[...] = block_ent

    @pl.when(step != 0)
    def _acc():
        counts_ref[...] += block_counts
        lse_ref[...] += block_lse
        ent_ref[...] += block_ent


@jax.jit
def _router(gate_weight, hidden_flat):
    n_tokens = hidden_flat.shape[0]
    grid = (n_tokens // TOKEN_BLOCK,)
    out_shapes = (
        jax.ShapeDtypeStruct((NUM_SELECTED, n_tokens), jnp.int32),
        jax.ShapeDtypeStruct((NUM_SELECTED, n_tokens), jnp.float32),
        jax.ShapeDtypeStruct((NUM_EXPERTS, 1), jnp.float32),
        jax.ShapeDtypeStruct((1, 1), jnp.float32),
        jax.ShapeDtypeStruct((1, 1), jnp.float32),
    )
    return pl.pallas_call(
        _router_block,
        grid=grid,
        in_specs=[
            pl.BlockSpec((NUM_EXPERTS, D_MODEL), lambda i: (0, 0)),
            pl.BlockSpec((TOKEN_BLOCK, D_MODEL), lambda i: (i, 0)),
        ],
        out_specs=(
            pl.BlockSpec((NUM_SELECTED, TOKEN_BLOCK), lambda i: (0, i)),
            pl.BlockSpec((NUM_SELECTED, TOKEN_BLOCK), lambda i: (0, i)),
            pl.BlockSpec((NUM_EXPERTS, 1), lambda i: (0, 0)),
            pl.BlockSpec((1, 1), lambda i: (0, 0)),
            pl.BlockSpec((1, 1), lambda i: (0, 0)),
        ),
        out_shape=out_shapes,
        compiler_params=pltpu.CompilerParams(
            dimension_semantics=("arbitrary",),
        ),
    )(gate_weight, hidden_flat)


def kernel(hidden_states, gate_weight):
    batch_size, seq_len, d_model = hidden_states.shape
    num_tokens = batch_size * seq_len
    hidden_flat = hidden_states.reshape(num_tokens, d_model)

    it, wt, counts2d, lse_sum, ent_sum = _router(gate_weight, hidden_flat)

    expert_counts = counts2d[:, 0]
    capacity = int(CAPACITY_FACTOR * num_tokens / NUM_EXPERTS * NUM_SELECTED)
    expert_overflow = jnp.sum(jnp.maximum(expert_counts - capacity, 0.0))
    capacity_overflow_pct = expert_overflow / num_tokens * 100.0
    z_loss = lse_sum[0, 0] / num_tokens * Z_LOSS_COEF
    gate_entropy = ent_sum[0, 0] / num_tokens
    expert_load_normalized = expert_counts / jnp.sum(expert_counts)
    ideal_load = 1.0 / NUM_EXPERTS
    expert_load_variance = jnp.mean((expert_load_normalized - ideal_load) ** 2)

    expert_indices = it.T.reshape(batch_size, seq_len, NUM_SELECTED)
    expert_weights = wt.T.reshape(batch_size, seq_len, NUM_SELECTED)
    routing_confidence = wt[0]
    return (expert_indices, expert_weights, expert_counts,
            capacity_overflow_pct, z_loss, gate_entropy,
            expert_load_variance, routing_confidence)
